# baseline (speedup 1.0000x reference)
import functools

import jax
import jax.numpy as jnp
from jax import lax
from jax.experimental import pallas as pl
from jax.experimental.pallas import tpu as pltpu
from jax.experimental.pallas import tpu_sc as plsc


@functools.lru_cache(maxsize=None)
def _build_gather(B, K, D):
    info = plsc.get_sparse_core_info()
    NC, NS = info.num_cores, info.num_subcores
    NW = NC * NS
    assert B % (8 * NW) == 0
    b_per_w = B // NW
    n_stagers = 8
    assert K % n_stagers == 0
    k_per_stager = K // n_stagers
    mesh = plsc.VectorSubcoreMesh(core_axis_name="c", subcore_axis_name="s")

    @functools.partial(
        pl.kernel,
        mesh=mesh,
        out_type=jax.ShapeDtypeStruct((B, D), jnp.float32),
        scratch_types=[
            pltpu.VMEM((b_per_w,), jnp.int32),
            pltpu.VMEM((b_per_w, D), jnp.float32),
            pltpu.VMEM_SHARED((K, D), jnp.float32),
            pltpu.SemaphoreType.DMA,
            pltpu.SemaphoreType.DMA,
        ],
    )
    def gather_kernel(y_hbm, table_hbm, out_hbm, idx_v, rows_v, table_sp,
                      gsem, wsem):
        sid = lax.axis_index("s")
        wid = sid * NC + lax.axis_index("c")
        base = wid * b_per_w

        @pl.when(sid < n_stagers)
        def _stage():
            r0 = sid * k_per_stager

            def _fire_stage(i, _):
                pltpu.make_async_copy(
                    table_hbm.at[r0 + i], table_sp.at[r0 + i], gsem
                ).start()
                return 0

            lax.fori_loop(0, k_per_stager, _fire_stage, 0, unroll=8)

            def _drain_stage(i, _):
                pltpu.make_async_copy(
                    table_hbm.at[r0 + i], table_sp.at[r0 + i], gsem
                ).wait()
                return 0

            lax.fori_loop(0, k_per_stager, _drain_stage, 0, unroll=8)

        plsc.subcore_barrier()
        pltpu.sync_copy(y_hbm.at[pl.ds(base, b_per_w)], idx_v)
        pltpu.async_copy(table_sp.at[idx_v], rows_v, gsem).wait()

        def _fire(i, _):
            pltpu.make_async_copy(
                rows_v.at[i], out_hbm.at[base + i], wsem
            ).start()
            return 0

        lax.fori_loop(0, b_per_w, _fire, 0, unroll=8)
        pltpu.make_async_copy(
            table_hbm.at[pl.ds(0, b_per_w)], rows_v, wsem
        ).wait()

    return gather_kernel


def kernel(y, codebook):
    (B,) = y.shape
    K, D = codebook.shape
    return _build_gather(B, K, D)(y, codebook)


# speedup vs baseline: 1.1186x; 1.1186x over previous
import functools

import jax
import jax.numpy as jnp
from jax import lax
from jax.experimental import pallas as pl
from jax.experimental.pallas import tpu as pltpu
from jax.experimental.pallas import tpu_sc as plsc


@functools.lru_cache(maxsize=None)
def _build_gather(B, K, D):
    info = plsc.get_sparse_core_info()
    NC, NS = info.num_cores, info.num_subcores
    NW = NC * NS
    assert B % (8 * NW) == 0
    b_per_w = B // NW
    DP = 128
    n_chunks = 4
    assert b_per_w % n_chunks == 0
    chunk = b_per_w // n_chunks
    assert chunk % 8 == 0
    rows_lo, rows_hi = 64, K - 15 * 64
    assert rows_hi > 0 and rows_hi % 8 == 0
    mesh = plsc.VectorSubcoreMesh(core_axis_name="c", subcore_axis_name="s")

    @functools.partial(
        pl.kernel,
        mesh=mesh,
        out_type=jax.ShapeDtypeStruct((B, DP), jnp.float32),
        scratch_types=[
            pltpu.VMEM((b_per_w,), jnp.int32),
            pltpu.VMEM((2, chunk, DP), jnp.float32),
            pltpu.VMEM_SHARED((K, DP), jnp.float32),
            pltpu.SemaphoreType.DMA,
            pltpu.SemaphoreType.DMA,
            pltpu.SemaphoreType.DMA,
        ],
    )
    def gather_kernel(y_hbm, table_hbm, out_hbm, idx_v, rows_v, table_sp,
                      isem, gsem, wsem):
        sid = lax.axis_index("s")
        wid = sid * NC + lax.axis_index("c")
        base = wid * b_per_w

        idx_cp = pltpu.make_async_copy(
            y_hbm.at[pl.ds(base, b_per_w)], idx_v, isem
        )
        idx_cp.start()

        @pl.when(sid < 15)
        def _stage_lo():
            pltpu.sync_copy(
                table_hbm.at[pl.ds(sid * rows_lo, rows_lo)],
                table_sp.at[pl.ds(sid * rows_lo, rows_lo)],
            )

        @pl.when(sid == 15)
        def _stage_hi():
            pltpu.sync_copy(
                table_hbm.at[pl.ds(15 * rows_lo, rows_hi)],
                table_sp.at[pl.ds(15 * rows_lo, rows_hi)],
            )

        plsc.subcore_barrier()
        idx_cp.wait()

        def _gather(c):
            return pltpu.make_async_copy(
                table_sp.at[idx_v.at[pl.ds(c * chunk, chunk)]],
                rows_v.at[c % 2],
                gsem,
            )

        def _write(c):
            return pltpu.make_async_copy(
                rows_v.at[c % 2],
                out_hbm.at[pl.ds(base + c * chunk, chunk)],
                wsem,
            )

        _gather(0).start()
        _gather(1).start()
        _gather(0).wait()
        _write(0).start()
        _gather(1).wait()
        _write(0).wait()
        _gather(2).start()
        _write(1).start()
        _gather(2).wait()
        _write(1).wait()
        _gather(3).start()
        _write(2).start()
        _gather(3).wait()
        _write(3).start()
        _write(2).wait()
        _write(3).wait()

    return gather_kernel


def kernel(y, codebook):
    (B,) = y.shape
    K, D = codebook.shape
    DP = 128
    table = jnp.concatenate(
        [codebook, jnp.zeros((K, DP - D), jnp.float32)], axis=1
    )
    out = _build_gather(B, K, D)(y, table)
    return out[:, :D]
